# baseline (speedup 1.0000x reference)
import functools

import jax
import jax.numpy as jnp
from jax import lax
from jax.experimental import pallas as pl
from jax.experimental.pallas import tpu as pltpu
from jax.experimental.pallas import tpu_sc as plsc

_LANES = 16


def _log_f32(x):
    xi = plsc.bitcast(x, jnp.int32)
    e = lax.shift_right_logical(xi, 23) - 127
    m = plsc.bitcast(
        (xi & jnp.int32(0x007FFFFF)) | jnp.int32(0x3F800000), jnp.float32
    )
    big = m > jnp.float32(1.41421356)
    m = jnp.where(big, m * jnp.float32(0.5), m)
    e = e + jnp.where(big, jnp.int32(1), jnp.int32(0))
    ef = e.astype(jnp.float32)
    f = m - jnp.float32(1.0)
    z = f * f
    p = jnp.float32(0.12485587)
    p = p * f + jnp.float32(-0.18030453)
    p = p * f + jnp.float32(0.20199712)
    p = p * f + jnp.float32(-0.24970138)
    p = p * f + jnp.float32(0.33331484)
    r = f + (f * z * p - jnp.float32(0.5) * z)
    r = r + ef * jnp.float32(0.6931471805599453)
    return jnp.where(x == jnp.float32(0.0), -jnp.inf, r)


def _dyn_gather16(vec, idx):
    dn = lax.GatherDimensionNumbers(
        offset_dims=(), collapsed_slice_dims=(0,), start_index_map=(0,)
    )
    return lax.gather(vec, idx[:, None], dn, slice_sizes=(1,),
                      mode=lax.GatherScatterMode.PROMISE_IN_BOUNDS)


def _build_sc_call(B, T, S):
    info = plsc.get_sparse_core_info()
    NC, NS = info.num_cores, info.num_subcores
    NW = NC * NS
    assert B % NW == 0 and B % 128 == 0 and S % 8 == 0 and T % 2 == 0
    rows_per_w = B // NW
    elems_per_w = rows_per_w * T
    two_row = 2 * T
    assert two_row % _LANES == 0
    n_two_row = rows_per_w // 2
    vecs_per_block = two_row // _LANES

    st_t = B * S
    st_shi = 8 * B
    st_bhi = 128 * 8

    mesh = plsc.VectorSubcoreMesh(core_axis_name="c", subcore_axis_name="s")

    @functools.partial(
        pl.kernel,
        mesh=mesh,
        out_type=jax.ShapeDtypeStruct((NW, 2, _LANES), jnp.float32),
        scratch_types=[
            pltpu.VMEM((elems_per_w,), jnp.int32),
            pltpu.VMEM((elems_per_w,), jnp.int32),
            pltpu.VMEM((elems_per_w,), jnp.float32),
            pltpu.VMEM((rows_per_w,), jnp.int32),
            pltpu.VMEM((2, _LANES), jnp.float32),
            [pltpu.SemaphoreType.DMA] * 4,
        ],
        compiler_params=pltpu.CompilerParams(
            needs_layout_passes=False,
            disable_bounds_checks=True,
            skip_device_barrier=True,
        ),
    )
    def sc_loss(target_hbm, logits_hbm, lengths_hbm, out_hbm,
                tgt_v, idx_v, val_v, len_v, res_v, sems):
        wid = lax.axis_index("s") * NC + lax.axis_index("c")
        base = wid * elems_per_w
        b_base = wid * rows_per_w

        pltpu.sync_copy(target_hbm.at[pl.ds(base, elems_per_w)], tgt_v)
        pltpu.sync_copy(lengths_hbm.at[pl.ds(b_base, rows_per_w)], len_v)

        lanes = lax.iota(jnp.int32, _LANES)

        def idx_body(r2, _):
            blk = r2 * two_row
            for j in range(vecs_per_block):
                t2 = j * _LANES + lanes
                in_row1 = t2 >= T
                t = jnp.where(in_row1, t2 - T, t2)
                b = b_base + r2 * 2 + jnp.where(in_row1, jnp.int32(1),
                                                jnp.int32(0))
                s = tgt_v[pl.ds(blk + j * _LANES, _LANES)]
                idx = (t * st_t
                       + lax.shift_right_logical(s, 3) * st_shi
                       + lax.shift_right_logical(b, 7) * st_bhi
                       + (s & 7) * 128 + (b & 127))
                idx_v[pl.ds(blk + j * _LANES, _LANES)] = idx
            return 0

        def row_body(r2, acc):
            blk = r2 * two_row
            g16 = (r2 >> 3) * _LANES
            lv = len_v[pl.ds(g16, _LANES)]
            sub0 = (r2 * 2) & (_LANES - 1)
            for j in range(vecs_per_block):
                t2 = j * _LANES + lanes
                in_row1 = t2 >= T
                t = jnp.where(in_row1, t2 - T, t2)
                sub = sub0 + jnp.where(in_row1, jnp.int32(1), jnp.int32(0))
                ln = _dyn_gather16(lv, sub)
                v = val_v[pl.ds(blk + j * _LANES, _LANES)]
                acc = acc + jnp.where(t < ln, _log_f32(v), jnp.float32(0.0))
            return acc

        n_chunks = 2
        r2_per_chunk = n_two_row // n_chunks
        elems_per_chunk = elems_per_w // n_chunks
        copies = []
        acc = jnp.zeros((_LANES,), jnp.float32)
        for c in range(n_chunks):
            lax.fori_loop(c * r2_per_chunk, (c + 1) * r2_per_chunk,
                          idx_body, 0)
            copies.append(pltpu.async_copy(
                logits_hbm.at[idx_v.at[pl.ds(c * elems_per_chunk,
                                             elems_per_chunk)]],
                val_v.at[pl.ds(c * elems_per_chunk, elems_per_chunk)],
                sems[c]))
            if c > 0:
                copies[c - 1].wait()
                acc = lax.fori_loop((c - 1) * r2_per_chunk,
                                    c * r2_per_chunk, row_body, acc)
        copies[n_chunks - 1].wait()
        acc = lax.fori_loop((n_chunks - 1) * r2_per_chunk, n_two_row,
                            row_body, acc)

        lsum = jnp.zeros((_LANES,), jnp.float32)
        for k in range(rows_per_w // _LANES):
            lsum = lsum + len_v[pl.ds(k * _LANES, _LANES)].astype(jnp.float32)

        res_v[0, :] = acc
        res_v[1, :] = lsum
        pltpu.sync_copy(res_v, out_hbm.at[wid])

    return sc_loss


def kernel(target, logits, lengths):
    B, T = target.shape
    S = logits.shape[-1]
    sc_loss = _build_sc_call(B, T, S)
    lg = jnp.transpose(logits, (1, 2, 0))
    lg = lg.reshape(T, S // 8, 8, B // 128, 128)
    lg = jnp.transpose(lg, (0, 1, 3, 2, 4))
    lg_flat = lg.reshape(-1)
    parts = sc_loss(
        target.reshape(-1).astype(jnp.int32),
        lg_flat,
        lengths.astype(jnp.int32),
    )
    log_sum = parts[:, 0, :].sum()
    len_sum = parts[:, 1, :].sum()
    return -log_sum / len_sum


# speedup vs baseline: 1.2596x; 1.2596x over previous
import functools

import jax
import jax.numpy as jnp
from jax import lax
from jax.experimental import pallas as pl
from jax.experimental.pallas import tpu as pltpu
from jax.experimental.pallas import tpu_sc as plsc

_LANES = 16


def _log_f32(x):
    xi = plsc.bitcast(x, jnp.int32)
    e = lax.shift_right_logical(xi, 23) - 127
    m = plsc.bitcast(
        (xi & jnp.int32(0x007FFFFF)) | jnp.int32(0x3F800000), jnp.float32
    )
    big = m > jnp.float32(1.41421356)
    m = jnp.where(big, m * jnp.float32(0.5), m)
    e = e + jnp.where(big, jnp.int32(1), jnp.int32(0))
    ef = e.astype(jnp.float32)
    f = m - jnp.float32(1.0)
    z = f * f
    p = jnp.float32(7.0376836292e-2)
    p = p * f + jnp.float32(-1.1514610310e-1)
    p = p * f + jnp.float32(1.1676998740e-1)
    p = p * f + jnp.float32(-1.2420140846e-1)
    p = p * f + jnp.float32(1.4249322787e-1)
    p = p * f + jnp.float32(-1.6668057665e-1)
    p = p * f + jnp.float32(2.0000714765e-1)
    p = p * f + jnp.float32(-2.4999993993e-1)
    p = p * f + jnp.float32(3.3333331174e-1)
    y = f * z * p
    y = y + ef * jnp.float32(-2.12194440e-4)
    y = y - jnp.float32(0.5) * z
    r = f + y
    r = r + ef * jnp.float32(0.693359375)
    return jnp.where(x == jnp.float32(0.0), -jnp.inf, r)


def _dyn_gather16(vec, idx):
    dn = lax.GatherDimensionNumbers(
        offset_dims=(), collapsed_slice_dims=(0,), start_index_map=(0,)
    )
    return lax.gather(vec, idx[:, None], dn, slice_sizes=(1,),
                      mode=lax.GatherScatterMode.PROMISE_IN_BOUNDS)


def _build_sc_call(B, T, S):
    info = plsc.get_sparse_core_info()
    NC, NS = info.num_cores, info.num_subcores
    NW = NC * NS
    assert B % NW == 0 and B % 128 == 0 and S % 8 == 0 and T % 2 == 0
    rows_per_w = B // NW
    elems_per_w = rows_per_w * T
    two_row = 2 * T
    assert two_row % _LANES == 0
    n_two_row = rows_per_w // 2
    vecs_per_block = two_row // _LANES

    st_t = B * S
    st_shi = 8 * B
    st_bhi = 128 * 8

    mesh = plsc.VectorSubcoreMesh(core_axis_name="c", subcore_axis_name="s")

    @functools.partial(
        pl.kernel,
        mesh=mesh,
        out_type=jax.ShapeDtypeStruct((NW, 2, _LANES), jnp.float32),
        scratch_types=[
            pltpu.VMEM((elems_per_w,), jnp.int32),
            pltpu.VMEM((elems_per_w,), jnp.int32),
            pltpu.VMEM((elems_per_w,), jnp.float32),
            pltpu.VMEM((rows_per_w,), jnp.int32),
            pltpu.VMEM((2, _LANES), jnp.float32),
            [pltpu.SemaphoreType.DMA] * 4,
        ],
        compiler_params=pltpu.CompilerParams(
            needs_layout_passes=False,
            disable_bounds_checks=True,
            skip_device_barrier=True,
        ),
    )
    def sc_loss(target_hbm, logits_hbm, lengths_hbm, out_hbm,
                tgt_v, idx_v, val_v, len_v, res_v, sems):
        wid = lax.axis_index("s") * NC + lax.axis_index("c")
        base = wid * elems_per_w
        b_base = wid * rows_per_w

        pltpu.sync_copy(target_hbm.at[pl.ds(base, elems_per_w)], tgt_v)
        pltpu.sync_copy(lengths_hbm.at[pl.ds(b_base, rows_per_w)], len_v)

        lanes = lax.iota(jnp.int32, _LANES)

        def idx_body(r2, _):
            blk = r2 * two_row
            for j in range(vecs_per_block):
                t2 = j * _LANES + lanes
                in_row1 = t2 >= T
                t = jnp.where(in_row1, t2 - T, t2)
                b = b_base + r2 * 2 + jnp.where(in_row1, jnp.int32(1),
                                                jnp.int32(0))
                s = tgt_v[pl.ds(blk + j * _LANES, _LANES)]
                idx = (t * st_t
                       + lax.shift_right_logical(s, 3) * st_shi
                       + lax.shift_right_logical(b, 7) * st_bhi
                       + (s & 7) * 128 + (b & 127))
                idx_v[pl.ds(blk + j * _LANES, _LANES)] = idx
            return 0

        def row_body(r2, acc):
            blk = r2 * two_row
            g16 = (r2 >> 3) * _LANES
            lv = len_v[pl.ds(g16, _LANES)]
            sub0 = (r2 * 2) & (_LANES - 1)
            for j in range(vecs_per_block):
                t2 = j * _LANES + lanes
                in_row1 = t2 >= T
                t = jnp.where(in_row1, t2 - T, t2)
                sub = sub0 + jnp.where(in_row1, jnp.int32(1), jnp.int32(0))
                ln = _dyn_gather16(lv, sub)
                v = val_v[pl.ds(blk + j * _LANES, _LANES)]
                acc = acc + jnp.where(t < ln, _log_f32(v), jnp.float32(0.0))
            return acc

        n_chunks = 2
        r2_per_chunk = n_two_row // n_chunks
        elems_per_chunk = elems_per_w // n_chunks
        copies = []
        acc = jnp.zeros((_LANES,), jnp.float32)
        for c in range(n_chunks):
            lax.fori_loop(c * r2_per_chunk, (c + 1) * r2_per_chunk,
                          idx_body, 0)
            copies.append(pltpu.async_copy(
                logits_hbm.at[idx_v.at[pl.ds(c * elems_per_chunk,
                                             elems_per_chunk)]],
                val_v.at[pl.ds(c * elems_per_chunk, elems_per_chunk)],
                sems[c]))
            if c > 0:
                copies[c - 1].wait()
                acc = lax.fori_loop((c - 1) * r2_per_chunk,
                                    c * r2_per_chunk, row_body, acc)
        copies[n_chunks - 1].wait()
        acc = lax.fori_loop((n_chunks - 1) * r2_per_chunk, n_two_row,
                            row_body, acc)

        lsum = jnp.zeros((_LANES,), jnp.float32)
        for k in range(rows_per_w // _LANES):
            lsum = lsum + len_v[pl.ds(k * _LANES, _LANES)].astype(jnp.float32)

        res_v[0, :] = acc
        res_v[1, :] = lsum
        pltpu.sync_copy(res_v, out_hbm.at[wid])

    return sc_loss


def kernel(target, logits, lengths):
    B, T = target.shape
    S = logits.shape[-1]
    sc_loss = _build_sc_call(B, T, S)
    lg = jnp.transpose(logits, (1, 2, 0))
    lg = lg.reshape(T, S // 8, 8, B // 128, 128)
    lg = jnp.transpose(lg, (0, 1, 3, 2, 4))
    lg_flat = lg.reshape(-1)
    parts = sc_loss(
        target.reshape(-1).astype(jnp.int32),
        lg_flat,
        lengths.astype(jnp.int32),
    )
    log_sum = parts[:, 0, :].sum()
    len_sum = parts[:, 1, :].sum()
    return -log_sum / len_sum
